# baseline (speedup 1.0000x reference)
import functools

import jax
import jax.numpy as jnp
from jax import lax
from jax.experimental import pallas as pl
from jax.experimental.pallas import tpu as pltpu
from jax.experimental.pallas import tpu_sc as plsc

N_NODES = 10000
N_EDGES = 320000
D = 128
FC_IN = 16

NC = 2
NS = 16
NW = NC * NS
EPW = N_EDGES // NW
B = 80
NBLK = EPW // B
RPT = N_NODES // NS


def _wmat_body(s_ref, a_ref, w_ref, o_ref):
    m = jax.lax.dot_general(
        s_ref[...], w_ref[...],
        dimension_numbers=(((1,), (0,)), ((), ())),
        preferred_element_type=jnp.float32,
    )
    o_ref[...] = m * a_ref[...]


def _edge_weights(edge_scalars, edge_attr, fc_w_scaled):
    BE = 2560
    return pl.pallas_call(
        _wmat_body,
        grid=(N_EDGES // BE,),
        in_specs=[
            pl.BlockSpec((BE, FC_IN), lambda i: (i, 0)),
            pl.BlockSpec((BE, 1), lambda i: (i, 0)),
            pl.BlockSpec((FC_IN, D), lambda i: (0, 0)),
        ],
        out_specs=pl.BlockSpec((BE, D), lambda i: (i, 0)),
        out_shape=jax.ShapeDtypeStruct((N_EDGES, D), jnp.float32),
    )(edge_scalars, edge_attr, fc_w_scaled)


IDEPTH = 4
XDEPTH = 2


def _sc_body(x_hbm, src_hbm, dst_hbm, w_hbm, zeros_hbm, out_hbm, *scratch):
    src_i = scratch[0:IDEPTH]
    dst_i = scratch[IDEPTH:2 * IDEPTH]
    x_v = scratch[2 * IDEPTH:2 * IDEPTH + XDEPTH]
    w_v = scratch[2 * IDEPTH + XDEPTH:2 * IDEPTH + 2 * XDEPTH]
    acc_sh = scratch[2 * IDEPTH + 2 * XDEPTH]
    isem = scratch[2 * IDEPTH + 2 * XDEPTH + 1:2 * IDEPTH + 2 * XDEPTH + 1 + IDEPTH]
    dsem = scratch[2 * IDEPTH + 2 * XDEPTH + 1 + IDEPTH:]

    c = lax.axis_index("c")
    s = lax.axis_index("s")
    wid = c * NS + s
    base = wid * EPW

    def issue_idx(i, b):
        off = base + i * B
        pltpu.async_copy(src_hbm.at[pl.ds(off, B)], src_i[b], isem[b])
        pltpu.async_copy(dst_hbm.at[pl.ds(off, B)], dst_i[b], isem[b])

    def wait_idx(b):
        pltpu.make_async_copy(src_hbm.at[pl.ds(0, B)], src_i[b], isem[b]).wait()
        pltpu.make_async_copy(dst_hbm.at[pl.ds(0, B)], dst_i[b], isem[b]).wait()

    def issue_data(i, ib, xb):
        pltpu.async_copy(x_hbm.at[src_i[ib]], x_v[xb], dsem[xb])
        pltpu.async_copy(w_hbm.at[pl.ds(base + i * B, B)], w_v[xb], dsem[xb])

    def wait_data(xb):
        pltpu.make_async_copy(x_hbm.at[pl.ds(0, B)], x_v[xb], dsem[xb]).wait()
        pltpu.make_async_copy(w_hbm.at[pl.ds(0, B)], w_v[xb], dsem[xb]).wait()

    pltpu.sync_copy(zeros_hbm.at[s], acc_sh.at[pl.ds(s * RPT, RPT)])

    for b in range(IDEPTH):
        issue_idx(b, b)
    for b in range(XDEPTH):
        wait_idx(b)
        issue_data(b, b, b)

    plsc.subcore_barrier()

    @pl.loop(0, NBLK, step=IDEPTH)
    def outer(g):
        for b in range(IDEPTH):
            i = g + b
            xb = b % XDEPTH

            @pl.when(i < NBLK)
            def _process():
                wait_data(xb)

                def mul_body(j, _):
                    for u in range(D // 16):
                        sl = pl.ds(u * 16, 16)
                        w_v[xb][j, sl] = w_v[xb][j, sl] * x_v[xb][j, sl]
                    return 0

                lax.fori_loop(0, B, mul_body, 0)
                pltpu.sync_copy(w_v[xb], acc_sh.at[dst_i[b]], add=True)

            @pl.when(i + XDEPTH < NBLK)
            def _next_data():
                wait_idx((b + XDEPTH) % IDEPTH)
                issue_data(i + XDEPTH, (b + XDEPTH) % IDEPTH, xb)

            @pl.when(i + IDEPTH < NBLK)
            def _next_idx():
                issue_idx(i + IDEPTH, b)

    plsc.subcore_barrier()
    pltpu.sync_copy(acc_sh.at[pl.ds(s * RPT, RPT)], out_hbm.at[c, s])


_sc_scatter = functools.partial(
    pl.kernel,
    out_type=pltpu.HBM((NC, NS, RPT, D), jnp.float32),
    mesh=plsc.VectorSubcoreMesh(core_axis_name="c", subcore_axis_name="s"),
    scratch_types=(
        [pltpu.VMEM((B,), jnp.int32) for _ in range(2 * IDEPTH)]
        + [pltpu.VMEM((B, D), jnp.float32) for _ in range(2 * XDEPTH)]
        + [pltpu.VMEM_SHARED((N_NODES, D), jnp.float32)]
        + [pltpu.SemaphoreType.DMA for _ in range(IDEPTH + XDEPTH)]
    ),
)(_sc_body)


def _red_body(p_ref, o_ref):
    o_ref[...] = p_ref[0] + p_ref[1]


def _reduce_partials(partials):
    out = pl.pallas_call(
        _red_body,
        grid=(NS,),
        in_specs=[pl.BlockSpec((NC, 1, RPT, D), lambda i: (0, i, 0, 0))],
        out_specs=pl.BlockSpec((1, RPT, D), lambda i: (i, 0, 0)),
        out_shape=jax.ShapeDtypeStruct((NS, RPT, D), jnp.float32),
    )(partials)
    return out.reshape(N_NODES, D)


def kernel(node_input, node_attr, edge_src, edge_dst, edge_attr, edge_scalars, fc_w):
    scale = 1.0 / (jnp.sqrt(jnp.float32(FC_IN)) * jnp.sqrt(jnp.float32(32.0)))
    w_all = _edge_weights(edge_scalars, edge_attr, fc_w * scale)
    zeros = jnp.zeros((NS, RPT, D), jnp.float32)
    partials = _sc_scatter(node_input, edge_src, edge_dst, w_all, zeros)
    return _reduce_partials(partials)


# speedup vs baseline: 1.5835x; 1.5835x over previous
import functools

import jax
import jax.numpy as jnp
from jax import lax
from jax.experimental import pallas as pl
from jax.experimental.pallas import tpu as pltpu
from jax.experimental.pallas import tpu_sc as plsc

N_NODES = 10000
N_EDGES = 320000
D = 128
FC_IN = 16

NC = 2
NS = 16
NW = NC * NS
EPW = N_EDGES // NW
B = 80
NBLK = EPW // B
RPT = N_NODES // NS


def _wmat_body(sT_ref, aT_ref, w_ref, o_ref):
    x = sT_ref[...] * aT_ref[...]
    o_ref[...] = jax.lax.dot_general(
        x, w_ref[...],
        dimension_numbers=(((0,), (0,)), ((), ())),
        preferred_element_type=jnp.float32,
    )


def _edge_weights(edge_scalars_t, edge_attr_t, fc_w_scaled):
    BE = 2560
    return pl.pallas_call(
        _wmat_body,
        grid=(N_EDGES // BE,),
        in_specs=[
            pl.BlockSpec((FC_IN, BE), lambda i: (0, i)),
            pl.BlockSpec((1, BE), lambda i: (0, i)),
            pl.BlockSpec((FC_IN, D), lambda i: (0, 0)),
        ],
        out_specs=pl.BlockSpec((BE, D), lambda i: (i, 0)),
        out_shape=jax.ShapeDtypeStruct((N_EDGES, D), jnp.float32),
    )(edge_scalars_t, edge_attr_t, fc_w_scaled)


IDEPTH = 4
XDEPTH = 2


def _sc_body(x_hbm, src_hbm, dst_hbm, w_hbm, zeros_hbm, out_hbm, *scratch):
    src_i = scratch[0:IDEPTH]
    dst_i = scratch[IDEPTH:2 * IDEPTH]
    x_v = scratch[2 * IDEPTH:2 * IDEPTH + XDEPTH]
    w_v = scratch[2 * IDEPTH + XDEPTH:2 * IDEPTH + 2 * XDEPTH]
    acc_sh = scratch[2 * IDEPTH + 2 * XDEPTH]
    isem = scratch[2 * IDEPTH + 2 * XDEPTH + 1:2 * IDEPTH + 2 * XDEPTH + 1 + IDEPTH]
    dsem = scratch[2 * IDEPTH + 2 * XDEPTH + 1 + IDEPTH:]

    c = lax.axis_index("c")
    s = lax.axis_index("s")
    wid = c * NS + s
    base = wid * EPW

    def issue_idx(i, b):
        off = base + i * B
        pltpu.async_copy(src_hbm.at[pl.ds(off, B)], src_i[b], isem[b])
        pltpu.async_copy(dst_hbm.at[pl.ds(off, B)], dst_i[b], isem[b])

    def wait_idx(b):
        pltpu.make_async_copy(src_hbm.at[pl.ds(0, B)], src_i[b], isem[b]).wait()
        pltpu.make_async_copy(dst_hbm.at[pl.ds(0, B)], dst_i[b], isem[b]).wait()

    def issue_data(i, ib, xb):
        pltpu.async_copy(x_hbm.at[src_i[ib]], x_v[xb], dsem[xb])
        pltpu.async_copy(w_hbm.at[pl.ds(base + i * B, B)], w_v[xb], dsem[xb])

    def wait_data(xb):
        pltpu.make_async_copy(x_hbm.at[pl.ds(0, B)], x_v[xb], dsem[xb]).wait()
        pltpu.make_async_copy(w_hbm.at[pl.ds(0, B)], w_v[xb], dsem[xb]).wait()

    pltpu.sync_copy(zeros_hbm.at[s], acc_sh.at[pl.ds(s * RPT, RPT)])

    for b in range(IDEPTH):
        issue_idx(b, b)
    for b in range(XDEPTH):
        wait_idx(b)
        issue_data(b, b, b)

    plsc.subcore_barrier()

    @pl.loop(0, NBLK, step=IDEPTH)
    def outer(g):
        for b in range(IDEPTH):
            i = g + b
            xb = b % XDEPTH

            @pl.when(i < NBLK)
            def _process():
                wait_data(xb)

                @plsc.parallel_loop(0, B, unroll=4)
                def mul_body(j):
                    for u in range(D // 16):
                        sl = pl.ds(u * 16, 16)
                        w_v[xb][j, sl] = w_v[xb][j, sl] * x_v[xb][j, sl]
                pltpu.sync_copy(w_v[xb], acc_sh.at[dst_i[b]], add=True)

            @pl.when(i + XDEPTH < NBLK)
            def _next_data():
                wait_idx((b + XDEPTH) % IDEPTH)
                issue_data(i + XDEPTH, (b + XDEPTH) % IDEPTH, xb)

            @pl.when(i + IDEPTH < NBLK)
            def _next_idx():
                issue_idx(i + IDEPTH, b)

    plsc.subcore_barrier()
    pltpu.sync_copy(acc_sh.at[pl.ds(s * RPT, RPT)], out_hbm.at[c, s])


_sc_scatter = functools.partial(
    pl.kernel,
    out_type=pltpu.HBM((NC, NS, RPT, D), jnp.float32),
    mesh=plsc.VectorSubcoreMesh(core_axis_name="c", subcore_axis_name="s"),
    scratch_types=(
        [pltpu.VMEM((B,), jnp.int32) for _ in range(2 * IDEPTH)]
        + [pltpu.VMEM((B, D), jnp.float32) for _ in range(2 * XDEPTH)]
        + [pltpu.VMEM_SHARED((N_NODES, D), jnp.float32)]
        + [pltpu.SemaphoreType.DMA for _ in range(IDEPTH + XDEPTH)]
    ),
)(_sc_body)


def _red_body(p_ref, o_ref):
    o_ref[...] = p_ref[0] + p_ref[1]


def _reduce_partials(partials):
    out = pl.pallas_call(
        _red_body,
        grid=(NS,),
        in_specs=[pl.BlockSpec((NC, 1, RPT, D), lambda i: (0, i, 0, 0))],
        out_specs=pl.BlockSpec((1, RPT, D), lambda i: (i, 0, 0)),
        out_shape=jax.ShapeDtypeStruct((NS, RPT, D), jnp.float32),
    )(partials)
    return out.reshape(N_NODES, D)


def kernel(node_input, node_attr, edge_src, edge_dst, edge_attr, edge_scalars, fc_w):
    scale = 1.0 / (jnp.sqrt(jnp.float32(FC_IN)) * jnp.sqrt(jnp.float32(32.0)))
    w_all = _edge_weights(edge_scalars.T, edge_attr.T, fc_w * scale)
    zeros = jnp.zeros((NS, RPT, D), jnp.float32)
    partials = _sc_scatter(node_input, edge_src, edge_dst, w_all, zeros)
    return _reduce_partials(partials)


# speedup vs baseline: 1.6032x; 1.0124x over previous
import functools

import jax
import jax.numpy as jnp
from jax import lax
from jax.experimental import pallas as pl
from jax.experimental.pallas import tpu as pltpu
from jax.experimental.pallas import tpu_sc as plsc

N_NODES = 10000
N_EDGES = 320000
D = 128
FC_IN = 16

NC = 2
NS = 16
NW = NC * NS
EPW = N_EDGES // NW
B = 80
NBLK = EPW // B
RPT = N_NODES // NS


def _wmat_body(sT_ref, aT_ref, w_ref, o_ref):
    x = sT_ref[...] * aT_ref[...]
    o_ref[...] = jax.lax.dot_general(
        x, w_ref[...],
        dimension_numbers=(((0,), (0,)), ((), ())),
        preferred_element_type=jnp.float32,
    )


def _edge_weights(edge_scalars_t, edge_attr_t, fc_w_scaled):
    BE = 2560
    return pl.pallas_call(
        _wmat_body,
        grid=(N_EDGES // BE,),
        in_specs=[
            pl.BlockSpec((FC_IN, BE), lambda i: (0, i)),
            pl.BlockSpec((1, BE), lambda i: (0, i)),
            pl.BlockSpec((FC_IN, D), lambda i: (0, 0)),
        ],
        out_specs=pl.BlockSpec((BE, D), lambda i: (i, 0)),
        out_shape=jax.ShapeDtypeStruct((N_EDGES, D), jnp.float32),
    )(edge_scalars_t, edge_attr_t, fc_w_scaled)


IDEPTH = 4
XDEPTH = 2


def _sc_body(x_hbm, src_hbm, dst_hbm, w_hbm, zeros_hbm, out_hbm, *scratch):
    src_i = scratch[0:IDEPTH]
    dst_i = scratch[IDEPTH:2 * IDEPTH]
    x_v = scratch[2 * IDEPTH:2 * IDEPTH + XDEPTH]
    w_v = scratch[2 * IDEPTH + XDEPTH:2 * IDEPTH + 2 * XDEPTH]
    acc_sh = scratch[2 * IDEPTH + 2 * XDEPTH]
    isem = scratch[2 * IDEPTH + 2 * XDEPTH + 1:2 * IDEPTH + 2 * XDEPTH + 1 + IDEPTH]
    dsem = scratch[2 * IDEPTH + 2 * XDEPTH + 1 + IDEPTH:]

    c = lax.axis_index("c")
    s = lax.axis_index("s")
    wid = c * NS + s
    base = wid * EPW

    def issue_idx(i, b):
        off = base + i * B
        pltpu.async_copy(src_hbm.at[pl.ds(off, B)], src_i[b], isem[b])
        pltpu.async_copy(dst_hbm.at[pl.ds(off, B)], dst_i[b], isem[b])

    def wait_idx(b):
        pltpu.make_async_copy(src_hbm.at[pl.ds(0, B)], src_i[b], isem[b]).wait()
        pltpu.make_async_copy(dst_hbm.at[pl.ds(0, B)], dst_i[b], isem[b]).wait()

    def issue_data(i, ib, xb):
        pltpu.async_copy(x_hbm.at[src_i[ib]], x_v[xb], dsem[xb])
        pltpu.async_copy(w_hbm.at[pl.ds(base + i * B, B)], w_v[xb], dsem[xb])

    def wait_data(xb):
        pltpu.make_async_copy(x_hbm.at[pl.ds(0, B)], x_v[xb], dsem[xb]).wait()
        pltpu.make_async_copy(w_hbm.at[pl.ds(0, B)], w_v[xb], dsem[xb]).wait()

    pltpu.sync_copy(zeros_hbm.at[s], acc_sh.at[pl.ds(s * RPT, RPT)])

    for b in range(IDEPTH):
        issue_idx(b, b)
    for b in range(XDEPTH):
        wait_idx(b)
        issue_data(b, b, b)

    plsc.subcore_barrier()

    @pl.loop(0, NBLK, step=IDEPTH)
    def outer(g):
        for b in range(IDEPTH):
            i = g + b
            xb = b % XDEPTH

            @pl.when(i < NBLK)
            def _process():
                wait_data(xb)

                def mul_body(j, _):
                    for u in range(D // 16):
                        sl = pl.ds(u * 16, 16)
                        w_v[xb][j, sl] = w_v[xb][j, sl] * x_v[xb][j, sl]
                    return 0

                lax.fori_loop(0, B, mul_body, 0)
                pltpu.sync_copy(w_v[xb], acc_sh.at[dst_i[b]], add=True)

            @pl.when(i + XDEPTH < NBLK)
            def _next_data():
                wait_idx((b + XDEPTH) % IDEPTH)
                issue_data(i + XDEPTH, (b + XDEPTH) % IDEPTH, xb)

            @pl.when(i + IDEPTH < NBLK)
            def _next_idx():
                issue_idx(i + IDEPTH, b)

    plsc.subcore_barrier()
    pltpu.sync_copy(acc_sh.at[pl.ds(s * RPT, RPT)], out_hbm.at[c, s])


_sc_scatter = functools.partial(
    pl.kernel,
    out_type=pltpu.HBM((NC, NS, RPT, D), jnp.float32),
    mesh=plsc.VectorSubcoreMesh(core_axis_name="c", subcore_axis_name="s"),
    scratch_types=(
        [pltpu.VMEM((B,), jnp.int32) for _ in range(2 * IDEPTH)]
        + [pltpu.VMEM((B, D), jnp.float32) for _ in range(2 * XDEPTH)]
        + [pltpu.VMEM_SHARED((N_NODES, D), jnp.float32)]
        + [pltpu.SemaphoreType.DMA for _ in range(IDEPTH + XDEPTH)]
    ),
)(_sc_body)


def _red_body(p_ref, o_ref):
    o_ref[...] = p_ref[0] + p_ref[1]


def _reduce_partials(partials):
    out = pl.pallas_call(
        _red_body,
        grid=(NS,),
        in_specs=[pl.BlockSpec((NC, 1, RPT, D), lambda i: (0, i, 0, 0))],
        out_specs=pl.BlockSpec((1, RPT, D), lambda i: (i, 0, 0)),
        out_shape=jax.ShapeDtypeStruct((NS, RPT, D), jnp.float32),
    )(partials)
    return out.reshape(N_NODES, D)


def kernel(node_input, node_attr, edge_src, edge_dst, edge_attr, edge_scalars, fc_w):
    scale = 1.0 / (jnp.sqrt(jnp.float32(FC_IN)) * jnp.sqrt(jnp.float32(32.0)))
    w_all = _edge_weights(edge_scalars.T, edge_attr.T, fc_w * scale)
    zeros = jnp.zeros((NS, RPT, D), jnp.float32)
    partials = _sc_scatter(node_input, edge_src, edge_dst, w_all, zeros)
    return _reduce_partials(partials)
